# Initial kernel scaffold; baseline (speedup 1.0000x reference)
import jax
import jax.numpy as jnp
from jax.experimental import pallas as pl


def kernel(x, edge_index, num_target, W_l, b_l, W_r):
    raise NotImplementedError("write your pallas kernel here")


# speedup vs baseline: 5.0106x; 5.0106x over previous
import functools

import jax
import jax.numpy as jnp
from jax import lax
from jax.experimental import pallas as pl
from jax.experimental.pallas import tpu as pltpu
from jax.experimental.pallas import tpu_sc as plsc

N = 10000
T = 2048
E = 320000
D = 128
O = 64

NC = 2
NS = 16
NW = NC * NS

C = 128
NCH = -(-E // (NW * C))
E_PAD = NW * C * NCH
EW = NCH * C
CW = 128
T_PAD = ((T + 1 + NS * 8 - 1) // (NS * 8)) * (NS * 8)
RT = T_PAD // NS


def _sc_accumulate(x, src_w, dst_w, zsum, zcnt):
    mesh = plsc.VectorSubcoreMesh(core_axis_name="c", subcore_axis_name="s",
                                  num_cores=NC, num_subcores=NS)

    @functools.partial(
        pl.kernel,
        out_type=(
            jax.ShapeDtypeStruct((NC, T_PAD, D), jnp.float32),
            jax.ShapeDtypeStruct((NC, T_PAD, CW), jnp.float32),
        ),
        mesh=mesh,
        scratch_types=[
            pltpu.VMEM((C,), jnp.int32),
            pltpu.VMEM((C,), jnp.int32),
            pltpu.VMEM((C, D), jnp.float32),
            pltpu.VMEM((C, CW), jnp.float32),
            pltpu.VMEM_SHARED((T_PAD, D), jnp.float32),
            pltpu.VMEM_SHARED((T_PAD, CW), jnp.float32),
            pltpu.SemaphoreType.DMA,
        ],
    )
    def body(x_hbm, src_hbm, dst_hbm, zsum_hbm, zcnt_hbm,
             sums_out, cnt_out, src_v, dst_v, gbuf, ones_v, acc_sh, cnt_sh,
             sem):
        ci = lax.axis_index("c")
        si = lax.axis_index("s")
        wid = ci * NS + si

        pltpu.sync_copy(zsum_hbm.at[pl.ds(si * RT, RT)],
                        acc_sh.at[pl.ds(si * RT, RT)])
        pltpu.sync_copy(zcnt_hbm.at[pl.ds(si * RT, RT)],
                        cnt_sh.at[pl.ds(si * RT, RT)])
        def fill(i, carry):
            ones_v[i // (CW // 16), pl.ds((i % (CW // 16)) * 16, 16)] = (
                jnp.ones((16,), jnp.float32))
            return carry

        lax.fori_loop(0, C * (CW // 16), fill, 0)
        plsc.subcore_barrier()

        def step(j, carry):
            pltpu.sync_copy(src_hbm.at[wid, j], src_v)
            pltpu.sync_copy(dst_hbm.at[wid, j], dst_v)
            pltpu.async_copy(x_hbm.at[src_v], gbuf, sem).wait()
            pltpu.sync_copy(gbuf, acc_sh.at[dst_v], add=True)
            pltpu.sync_copy(ones_v, cnt_sh.at[dst_v], add=True)
            return carry

        lax.fori_loop(0, NCH, step, 0)
        plsc.subcore_barrier()

        pltpu.sync_copy(acc_sh.at[pl.ds(si * RT, RT)],
                        sums_out.at[ci, pl.ds(si * RT, RT)])
        pltpu.sync_copy(cnt_sh.at[pl.ds(si * RT, RT)],
                        cnt_out.at[ci, pl.ds(si * RT, RT)])

    return body(x, src_w, dst_w, zsum, zcnt)


def _tc_combine(sums_ref, cnt_ref, xt_ref, wl_ref, bl_ref, wr_ref, out_ref):
    s = sums_ref[0][:T] + sums_ref[1][:T]
    c = cnt_ref[0][:T, 0:1] + cnt_ref[1][:T, 0:1]
    mean = s / jnp.maximum(c, 1.0)
    h = lax.dot_general(mean, wl_ref[...],
                        (((1,), (1,)), ((), ())),
                        preferred_element_type=jnp.float32)
    h = h + bl_ref[...]
    h = h + lax.dot_general(xt_ref[...], wr_ref[...],
                            (((1,), (1,)), ((), ())),
                            preferred_element_type=jnp.float32)
    m = jnp.max(h, axis=-1, keepdims=True)
    e = h - m
    lse = jnp.log(jnp.sum(jnp.exp(e), axis=-1, keepdims=True))
    out_ref[...] = e - lse


def kernel(x, edge_index, num_target, W_l, b_l, W_r):
    del num_target
    src = edge_index[0]
    dst = edge_index[1]
    pad = E_PAD - E
    src_w = jnp.concatenate(
        [src, jnp.zeros((pad,), jnp.int32)]).reshape(NW, NCH, C)
    dst_w = jnp.concatenate(
        [dst, jnp.full((pad,), T, jnp.int32)]).reshape(NW, NCH, C)
    zsum = jnp.zeros((T_PAD, D), jnp.float32)
    zcnt = jnp.zeros((T_PAD, CW), jnp.float32)
    sums, cnt = _sc_accumulate(x, src_w, dst_w, zsum, zcnt)

    out = pl.pallas_call(
        _tc_combine,
        out_shape=jax.ShapeDtypeStruct((T, O), jnp.float32),
    )(sums, cnt, x[:T], W_l, b_l.reshape(1, O), W_r)
    return out
